# Initial kernel scaffold; baseline (speedup 1.0000x reference)
import jax
import jax.numpy as jnp
from jax.experimental import pallas as pl


def kernel(in_feat, edge_index, W_self1, W_neigh1, b1, W_self2, W_neigh2, b2, W_lin, b_lin):
    raise NotImplementedError("write your pallas kernel here")


# speedup vs baseline: 5.1005x; 5.1005x over previous
import functools

import jax
import jax.numpy as jnp
from jax import lax
from jax.experimental import pallas as pl
from jax.experimental.pallas import tpu as pltpu
from jax.experimental.pallas import tpu_sc as plsc

N = 10000
E = 320000
D_IN = 128
D_H = 128
D_OUT = 64

NC = 2
NS = 16
NW = NC * NS

CH = 128
EPW = ((E + NW * CH - 1) // (NW * CH)) * CH
E_PAD = EPW * NW
N_PAD = 10240
RPT = N_PAD // NS
DEGW = 16


def _make_sc_agg(D, with_deg):
    mesh = plsc.VectorSubcoreMesh(core_axis_name="c", subcore_axis_name="s")
    out_type = [jax.ShapeDtypeStruct((NC, N_PAD, D), jnp.float32)]
    scratch = [
        pltpu.VMEM((64, D), jnp.float32),
        pltpu.VMEM((CH,), jnp.int32),
        pltpu.VMEM((CH,), jnp.int32),
        pltpu.VMEM((CH, D), jnp.float32),
        pltpu.VMEM_SHARED((N_PAD, D), jnp.float32),
        pltpu.SemaphoreType.DMA,
    ]
    if with_deg:
        out_type.append(jax.ShapeDtypeStruct((NC, N_PAD, DEGW), jnp.float32))
        scratch += [
            pltpu.VMEM((RPT, DEGW), jnp.float32),
            pltpu.VMEM((CH, DEGW), jnp.float32),
            pltpu.VMEM_SHARED((N_PAD, DEGW), jnp.float32),
        ]

    def body(table_hbm, edge_hbm, *refs):
        if with_deg:
            (agg_out, deg_out, zacc, src_v, dst_v, rows_v, acc_sh, sem,
             zdeg, ones_v, deg_sh) = refs
        else:
            (agg_out, zacc, src_v, dst_v, rows_v, acc_sh, sem) = refs
        cid = lax.axis_index("c")
        sid = lax.axis_index("s")
        wid = cid * NS + sid

        zero16 = jnp.zeros((16,), jnp.float32)

        def zacc_row(r, carry):
            for k in range(D // 16):
                zacc[r, pl.ds(k * 16, 16)] = zero16
            return carry
        lax.fori_loop(0, 64, zacc_row, 0)

        if with_deg:
            one16 = jnp.ones((16,), jnp.float32)

            def zdeg_row(r, carry):
                zdeg[r, :] = zero16
                return carry
            lax.fori_loop(0, RPT, zdeg_row, 0)

            def ones_row(r, carry):
                ones_v[r, :] = one16
                return carry
            lax.fori_loop(0, CH, ones_row, 0)

        def zcopy(j, carry):
            pltpu.sync_copy(zacc, acc_sh.at[pl.ds(sid * RPT + j * 64, 64)])
            return carry
        lax.fori_loop(0, RPT // 64, zcopy, 0)
        if with_deg:
            pltpu.sync_copy(zdeg, deg_sh.at[pl.ds(sid * RPT, RPT)])

        plsc.subcore_barrier()

        def chunk(g, carry):
            base = wid * EPW + g * CH
            pltpu.sync_copy(edge_hbm.at[0, pl.ds(base, CH)], src_v)
            pltpu.sync_copy(edge_hbm.at[1, pl.ds(base, CH)], dst_v)
            pltpu.async_copy(table_hbm.at[src_v], rows_v, sem).wait()
            pltpu.sync_copy(rows_v, acc_sh.at[dst_v], add=True)
            if with_deg:
                pltpu.sync_copy(ones_v, deg_sh.at[dst_v], add=True)
            return carry
        lax.fori_loop(0, EPW // CH, chunk, 0)

        plsc.subcore_barrier()

        def wcopy(j, carry):
            off = sid * RPT + j * 64
            pltpu.sync_copy(acc_sh.at[pl.ds(off, 64)], zacc)
            pltpu.sync_copy(zacc, agg_out.at[cid, pl.ds(off, 64)])
            return carry
        lax.fori_loop(0, RPT // 64, wcopy, 0)
        if with_deg:
            pltpu.sync_copy(deg_sh.at[pl.ds(sid * RPT, RPT)], zdeg)
            pltpu.sync_copy(zdeg, deg_out.at[cid, pl.ds(sid * RPT, RPT)])

    return pl.kernel(
        body, out_type=out_type, mesh=mesh, scratch_types=scratch,
        compiler_params=pltpu.CompilerParams(use_tc_tiling_on_sc=False))


_sc_agg1 = _make_sc_agg(D_IN, with_deg=True)
_sc_agg2 = _make_sc_agg(D_OUT, with_deg=False)


BN = 1000


def _tc1_body(x_ref, agg_ref, degp_ref, ws1_ref, wn1_ref, b1_ref,
              ws2_ref, wn2_ref, b2_ref, wl_ref, blin_ref,
              h1p_ref, h1s_ref):
    x = x_ref[...]
    agg = agg_ref[0] + agg_ref[1]
    dp = degp_ref[...]
    deg = dp[0, :, 0] + dp[1, :, 0]
    inv = 1.0 / jnp.maximum(deg, 1.0)
    hn = agg * inv[:, None]
    dn = (((1,), (1,)), ((), ()))
    h1 = lax.dot_general(x, ws1_ref[...], dn)
    h1 = h1 + lax.dot_general(hn, wn1_ref[...], dn)
    h1 = jnp.maximum(h1 + b1_ref[...], 0.0)
    wl = wl_ref[...]
    wc_n = jnp.dot(wl, wn2_ref[...])
    wc_s = jnp.dot(wl, ws2_ref[...])
    bc = lax.dot_general(b2_ref[...], wl, dn) + blin_ref[...]
    h1p_ref[...] = lax.dot_general(h1, wc_n, dn)
    h1s_ref[...] = lax.dot_general(h1, wc_s, dn) + bc


def _tc1(x, agg, degp, ws1, wn1, b1, ws2, wn2, b2, wl, blin):
    grid = (N // BN,)
    return pl.pallas_call(
        _tc1_body,
        grid=grid,
        in_specs=[
            pl.BlockSpec((BN, D_IN), lambda i: (i, 0)),
            pl.BlockSpec((NC, BN, D_IN), lambda i: (0, i, 0)),
            pl.BlockSpec((NC, BN, DEGW), lambda i: (0, i, 0)),
            pl.BlockSpec((D_H, D_IN), lambda i: (0, 0)),
            pl.BlockSpec((D_H, D_IN), lambda i: (0, 0)),
            pl.BlockSpec((1, D_H), lambda i: (0, 0)),
            pl.BlockSpec((D_OUT, D_H), lambda i: (0, 0)),
            pl.BlockSpec((D_OUT, D_H), lambda i: (0, 0)),
            pl.BlockSpec((1, D_OUT), lambda i: (0, 0)),
            pl.BlockSpec((D_OUT, D_OUT), lambda i: (0, 0)),
            pl.BlockSpec((1, D_OUT), lambda i: (0, 0)),
        ],
        out_specs=[
            pl.BlockSpec((BN, D_OUT), lambda i: (i, 0)),
            pl.BlockSpec((BN, D_OUT), lambda i: (i, 0)),
        ],
        out_shape=[
            jax.ShapeDtypeStruct((N, D_OUT), jnp.float32),
            jax.ShapeDtypeStruct((N, D_OUT), jnp.float32),
        ],
    )(x, agg, degp, ws1, wn1, b1, ws2, wn2, b2, wl, blin)


def _tc2_body(h1s_ref, agg2_ref, degp_ref, out_ref):
    dp = degp_ref[...]
    deg = dp[0, :, 0] + dp[1, :, 0]
    inv = 1.0 / jnp.maximum(deg, 1.0)
    q = agg2_ref[0] + agg2_ref[1]
    out_ref[...] = h1s_ref[...] + q * inv[:, None]


def _tc2(h1s, agg2, degp):
    grid = (N // BN,)
    return pl.pallas_call(
        _tc2_body,
        grid=grid,
        in_specs=[
            pl.BlockSpec((BN, D_OUT), lambda i: (i, 0)),
            pl.BlockSpec((NC, BN, D_OUT), lambda i: (0, i, 0)),
            pl.BlockSpec((NC, BN, DEGW), lambda i: (0, i, 0)),
        ],
        out_specs=pl.BlockSpec((BN, D_OUT), lambda i: (i, 0)),
        out_shape=jax.ShapeDtypeStruct((N, D_OUT), jnp.float32),
    )(h1s, agg2, degp)


def kernel(in_feat, edge_index, W_self1, W_neigh1, b1, W_self2, W_neigh2,
           b2, W_lin, b_lin):
    pad = E_PAD - E
    pad_edges = jnp.stack([
        jnp.zeros((pad,), jnp.int32),
        jnp.full((pad,), N, jnp.int32),
    ])
    epad = jnp.concatenate([edge_index, pad_edges], axis=1)

    agg1, degp = _sc_agg1(in_feat, epad)
    h1p, h1s = _tc1(in_feat, agg1, degp, W_self1, W_neigh1,
                    b1.reshape(1, -1), W_self2, W_neigh2,
                    b2.reshape(1, -1), W_lin, b_lin.reshape(1, -1))
    (agg2,) = _sc_agg2(h1p, epad)
    return _tc2(h1s, agg2, degp)
